# Initial kernel scaffold; baseline (speedup 1.0000x reference)
import jax
import jax.numpy as jnp
from jax.experimental import pallas as pl


def kernel(x, edge_index, batch, emb_table, W1_0, b1_0, W2_0, b2_0, gamma_0, beta_0, W1_1, b1_1, W2_1, b2_1, gamma_1, beta_1, W1_2, b1_2, W2_2, b2_2, gamma_2, beta_2, Wc, bc):
    raise NotImplementedError("write your pallas kernel here")


# speedup vs baseline: 5.3527x; 5.3527x over previous
import functools

import jax
import jax.numpy as jnp
from jax import lax
from jax.experimental import pallas as pl
from jax.experimental.pallas import tpu as pltpu
from jax.experimental.pallas import tpu_sc as plsc

N = 10000
E = 320000
EMB = 128
G = 64
NCOORD3 = 273 * 3

NC = 2
NS = 16
NW = NC * NS
CH = 128
NCHUNK = E // CH
NP = 10240
ROWS_PER_TILE = NP // NS
HK = 80
NHCH = N // HK

_mesh = plsc.VectorSubcoreMesh(core_axis_name="c", subcore_axis_name="s")


@functools.partial(
    pl.kernel,
    out_type=jax.ShapeDtypeStruct((N, EMB), jnp.float32),
    mesh=_mesh,
    scratch_types=[
        pltpu.VMEM((HK,), jnp.int32),
        pltpu.VMEM((HK, EMB), jnp.float32),
        pltpu.SemaphoreType.DMA,
    ],
)
def _emb_gather(x_hbm, emb_hbm, out_hbm, xg_v, rows_v, sem):
    c = lax.axis_index("c")
    s = lax.axis_index("s")
    w = s * NC + c
    for k in range(4):
        ch = w * 4 + k

        @pl.when(ch < NHCH)
        def _():
            base = ch * HK
            pltpu.sync_copy(x_hbm.at[pl.ds(base, HK)], xg_v)
            pltpu.async_copy(emb_hbm.at[xg_v], rows_v, sem).wait()
            pltpu.sync_copy(rows_v, out_hbm.at[pl.ds(base, HK)])


@functools.partial(
    pl.kernel,
    out_type=jax.ShapeDtypeStruct((NC, NP, EMB), jnp.float32),
    mesh=_mesh,
    scratch_types=[
        pltpu.VMEM((CH,), jnp.int32),
        pltpu.VMEM((CH,), jnp.int32),
        pltpu.VMEM((CH, EMB), jnp.float32),
        pltpu.VMEM_SHARED((NP, EMB), jnp.float32),
        pltpu.SemaphoreType.DMA,
    ],
)
def _edge_agg(ei_hbm, h_hbm, zeros_hbm, out_hbm, src_v, dst_v, rows_v, agg_sh, sem):
    c = lax.axis_index("c")
    s = lax.axis_index("s")
    w = s * NC + c

    pltpu.sync_copy(zeros_hbm, agg_sh.at[pl.ds(s * ROWS_PER_TILE, ROWS_PER_TILE)])
    plsc.subcore_barrier()

    n_i = (NCHUNK - 1 - w) // NW + 1

    def body(i, carry):
        off = (w + NW * i) * CH
        pltpu.sync_copy(ei_hbm.at[0, pl.ds(off, CH)], src_v)
        pltpu.sync_copy(ei_hbm.at[1, pl.ds(off, CH)], dst_v)
        pltpu.async_copy(h_hbm.at[src_v], rows_v, sem).wait()
        pltpu.sync_copy(rows_v, agg_sh.at[dst_v], add=True)
        return carry

    lax.fori_loop(0, n_i, body, 0)
    plsc.subcore_barrier()

    pltpu.sync_copy(
        agg_sh.at[pl.ds(s * ROWS_PER_TILE, ROWS_PER_TILE)],
        out_hbm.at[c, pl.ds(s * ROWS_PER_TILE, ROWS_PER_TILE)],
    )


def _mlp_body(h_ref, a_ref, w1_ref, b1_ref, w2_ref, b2_ref, g_ref,
              bt_ref, o_ref, *, leaky):
    z = h_ref[...] + a_ref[0, 0:N, :] + a_ref[1, 0:N, :]
    z = jnp.dot(z, w1_ref[...], preferred_element_type=jnp.float32) + b1_ref[...]
    z = jnp.maximum(z, 0.0)
    z = jnp.dot(z, w2_ref[...], preferred_element_type=jnp.float32) + b2_ref[...]
    mu = jnp.mean(z, axis=0, keepdims=True)
    d = z - mu
    var = jnp.mean(d * d, axis=0, keepdims=True)
    zn = g_ref[...] * d * lax.rsqrt(var + 1e-5) + bt_ref[...]
    if leaky:
        zn = jnp.where(zn > 0, zn, 0.01 * zn)
    o_ref[...] = zn


def _final_body(h_ref, a_ref, w1_ref, b1_ref, w2_ref, b2_ref, g_ref,
                bt_ref, batch_ref, wc_ref, bc_ref, o_ref):
    z = h_ref[...] + a_ref[0, 0:N, :] + a_ref[1, 0:N, :]
    z = jnp.dot(z, w1_ref[...], preferred_element_type=jnp.float32) + b1_ref[...]
    z = jnp.maximum(z, 0.0)
    z = jnp.dot(z, w2_ref[...], preferred_element_type=jnp.float32) + b2_ref[...]
    mu = jnp.mean(z, axis=0, keepdims=True)
    d = z - mu
    var = jnp.mean(d * d, axis=0, keepdims=True)
    zn = g_ref[...] * d * lax.rsqrt(var + 1e-5) + bt_ref[...]

    onehot = jnp.where(
        batch_ref[...] == lax.broadcasted_iota(jnp.int32, (N, G), 1), 1.0, 0.0)
    cnt = jnp.maximum(jnp.sum(onehot, axis=0, keepdims=True), 1.0)
    oh_n = onehot / cnt
    rep = lax.dot_general(oh_n, zn, (((0,), (0,)), ((), ())),
                          preferred_element_type=jnp.float32)
    o_ref[...] = jnp.dot(rep, wc_ref[...],
                         preferred_element_type=jnp.float32) + bc_ref[...]


def _mlp_call(h, a, w1, b1, w2, b2, g, bt, leaky):
    return pl.pallas_call(
        functools.partial(_mlp_body, leaky=leaky),
        out_shape=jax.ShapeDtypeStruct((N, EMB), jnp.float32),
    )(h, a, w1, b1, w2, b2, g, bt)


def _final_call(h, a, w1, b1, w2, b2, g, bt, batch, wc, bc):
    return pl.pallas_call(
        _final_body,
        out_shape=jax.ShapeDtypeStruct((G, NCOORD3), jnp.float32),
    )(h, a, w1, b1, w2, b2, g, bt, batch, wc, bc)


def kernel(x, edge_index, batch, emb_table, W1_0, b1_0, W2_0, b2_0, gamma_0,
           beta_0, W1_1, b1_1, W2_1, b2_1, gamma_1, beta_1, W1_2, b1_2, W2_2,
           b2_2, gamma_2, beta_2, Wc, bc):
    zeros = jnp.zeros((ROWS_PER_TILE, EMB), jnp.float32)
    batch2 = batch.reshape(N, 1)

    h = _emb_gather(x.reshape(N), emb_table)

    params = [
        (W1_0, b1_0.reshape(1, EMB), W2_0, b2_0.reshape(1, EMB),
         gamma_0.reshape(1, EMB), beta_0.reshape(1, EMB)),
        (W1_1, b1_1.reshape(1, EMB), W2_1, b2_1.reshape(1, EMB),
         gamma_1.reshape(1, EMB), beta_1.reshape(1, EMB)),
        (W1_2, b1_2.reshape(1, EMB), W2_2, b2_2.reshape(1, EMB),
         gamma_2.reshape(1, EMB), beta_2.reshape(1, EMB)),
    ]

    for l, (w1, b1, w2, b2, g, bt) in enumerate(params):
        agg = _edge_agg(edge_index, h, zeros)
        if l < 2:
            h = _mlp_call(h, agg, w1, b1, w2, b2, g, bt, leaky=True)
        else:
            coords = _final_call(h, agg, w1, b1, w2, b2, g, bt,
                                 batch2, Wc, bc.reshape(1, NCOORD3))
    return coords.reshape(-1, 3)


# speedup vs baseline: 10.9562x; 2.0469x over previous
import functools

import jax
import jax.numpy as jnp
from jax import lax
from jax.experimental import pallas as pl
from jax.experimental.pallas import tpu as pltpu
from jax.experimental.pallas import tpu_sc as plsc

N = 10000
E = 320000
EMB = 128
G = 64
NCOORD3 = 273 * 3

NC = 2
NS = 16
NW = NC * NS
CH = 128
NCHUNK = E // CH
NP = 10240
ROWS_PER_TILE = NP // NS
HK = 80
NHCH = N // HK

_mesh = plsc.VectorSubcoreMesh(core_axis_name="c", subcore_axis_name="s")


@functools.partial(
    pl.kernel,
    out_type=jax.ShapeDtypeStruct((N, EMB), jnp.float32),
    mesh=_mesh,
    scratch_types=[
        pltpu.VMEM((HK,), jnp.int32),
        pltpu.VMEM((HK, EMB), jnp.float32),
        pltpu.SemaphoreType.DMA,
    ],
)
def _emb_gather(x_hbm, emb_hbm, out_hbm, xg_v, rows_v, sem):
    c = lax.axis_index("c")
    s = lax.axis_index("s")
    w = s * NC + c
    for k in range(4):
        ch = w * 4 + k

        @pl.when(ch < NHCH)
        def _():
            base = ch * HK
            pltpu.sync_copy(x_hbm.at[pl.ds(base, HK)], xg_v)
            pltpu.async_copy(emb_hbm.at[xg_v], rows_v, sem).wait()
            pltpu.sync_copy(rows_v, out_hbm.at[pl.ds(base, HK)])


NBUF = 4
NRB = 2


@functools.partial(
    pl.kernel,
    out_type=jax.ShapeDtypeStruct((NC, NP, EMB), jnp.float32),
    mesh=_mesh,
    scratch_types=[
        pltpu.VMEM((NBUF, CH), jnp.int32),
        pltpu.VMEM((NBUF, CH), jnp.int32),
        pltpu.VMEM((NRB, CH, EMB), jnp.float32),
        pltpu.VMEM_SHARED((NP, EMB), jnp.float32),
        pltpu.SemaphoreType.DMA((NBUF,)),
        pltpu.SemaphoreType.DMA((NBUF,)),
        pltpu.SemaphoreType.DMA((NRB,)),
        pltpu.SemaphoreType.DMA((NRB,)),
    ],
)
def _edge_agg(ei_hbm, h_hbm, zeros_hbm, out_hbm, srcb, dstb, rows, agg_sh,
              sem_si, sem_di, sem_g, sem_s):
    c = lax.axis_index("c")
    s = lax.axis_index("s")
    w = s * NC + c

    pltpu.sync_copy(zeros_hbm, agg_sh.at[pl.ds(s * ROWS_PER_TILE, ROWS_PER_TILE)])
    plsc.subcore_barrier()

    n_i = (NCHUNK - 1 - w) // NW + 1

    def off(i):
        return (w + NW * i) * CH

    def idx_desc(i, b):
        return (
            pltpu.make_async_copy(ei_hbm.at[0, pl.ds(off(i), CH)], srcb.at[b],
                                  sem_si.at[b]),
            pltpu.make_async_copy(ei_hbm.at[1, pl.ds(off(i), CH)], dstb.at[b],
                                  sem_di.at[b]),
        )

    def start_idx(i):
        b = lax.rem(i, NBUF)
        d0, d1 = idx_desc(i, b)
        d0.start()
        d1.start()

    def wait_idx(i):
        b = lax.rem(i, NBUF)
        d0, d1 = idx_desc(i, b)
        d0.wait()
        d1.wait()

    def gather_desc(i):
        b = lax.rem(i, NBUF)
        r = lax.rem(i, NRB)
        return pltpu.make_async_copy(h_hbm.at[srcb.at[b]], rows.at[r],
                                     sem_g.at[r])

    def scatter_desc(i):
        b = lax.rem(i, NBUF)
        r = lax.rem(i, NRB)
        return pltpu.make_async_copy(rows.at[r], agg_sh.at[dstb.at[b]],
                                     sem_s.at[r])

    start_idx(0)
    start_idx(1)
    wait_idx(0)
    gather_desc(0).start()

    def body(i, carry):
        @pl.when(i >= 1)
        def _():
            scatter_desc(i - 1).wait()

        @pl.when(i + 2 < n_i)
        def _():
            start_idx(i + 2)

        @pl.when(i + 1 < n_i)
        def _():
            wait_idx(i + 1)
            gather_desc(i + 1).start()

        gather_desc(i).wait()
        b = lax.rem(i, NBUF)
        r = lax.rem(i, NRB)
        pltpu.async_copy(rows.at[r], agg_sh.at[dstb.at[b]], sem_s.at[r],
                         add=True)
        return carry

    lax.fori_loop(0, n_i, body, 0)
    scatter_desc(n_i - 1).wait()

    plsc.subcore_barrier()
    pltpu.sync_copy(
        agg_sh.at[pl.ds(s * ROWS_PER_TILE, ROWS_PER_TILE)],
        out_hbm.at[c, pl.ds(s * ROWS_PER_TILE, ROWS_PER_TILE)],
    )


def _mlp_body(h_ref, a_ref, w1_ref, b1_ref, w2_ref, b2_ref, g_ref,
              bt_ref, o_ref, *, leaky):
    z = h_ref[...] + a_ref[0, 0:N, :] + a_ref[1, 0:N, :]
    z = jnp.dot(z, w1_ref[...], preferred_element_type=jnp.float32) + b1_ref[...]
    z = jnp.maximum(z, 0.0)
    z = jnp.dot(z, w2_ref[...], preferred_element_type=jnp.float32) + b2_ref[...]
    mu = jnp.mean(z, axis=0, keepdims=True)
    d = z - mu
    var = jnp.mean(d * d, axis=0, keepdims=True)
    zn = g_ref[...] * d * lax.rsqrt(var + 1e-5) + bt_ref[...]
    if leaky:
        zn = jnp.where(zn > 0, zn, 0.01 * zn)
    o_ref[...] = zn


def _final_body(h_ref, a_ref, w1_ref, b1_ref, w2_ref, b2_ref, g_ref,
                bt_ref, batch_ref, wc_ref, bc_ref, o_ref):
    z = h_ref[...] + a_ref[0, 0:N, :] + a_ref[1, 0:N, :]
    z = jnp.dot(z, w1_ref[...], preferred_element_type=jnp.float32) + b1_ref[...]
    z = jnp.maximum(z, 0.0)
    z = jnp.dot(z, w2_ref[...], preferred_element_type=jnp.float32) + b2_ref[...]
    mu = jnp.mean(z, axis=0, keepdims=True)
    d = z - mu
    var = jnp.mean(d * d, axis=0, keepdims=True)
    zn = g_ref[...] * d * lax.rsqrt(var + 1e-5) + bt_ref[...]

    onehot = jnp.where(
        batch_ref[...] == lax.broadcasted_iota(jnp.int32, (N, G), 1), 1.0, 0.0)
    cnt = jnp.maximum(jnp.sum(onehot, axis=0, keepdims=True), 1.0)
    oh_n = onehot / cnt
    rep = lax.dot_general(oh_n, zn, (((0,), (0,)), ((), ())),
                          preferred_element_type=jnp.float32)
    o_ref[...] = jnp.dot(rep, wc_ref[...],
                         preferred_element_type=jnp.float32) + bc_ref[...]


def _mlp_call(h, a, w1, b1, w2, b2, g, bt, leaky):
    return pl.pallas_call(
        functools.partial(_mlp_body, leaky=leaky),
        out_shape=jax.ShapeDtypeStruct((N, EMB), jnp.float32),
    )(h, a, w1, b1, w2, b2, g, bt)


def _final_call(h, a, w1, b1, w2, b2, g, bt, batch, wc, bc):
    return pl.pallas_call(
        _final_body,
        out_shape=jax.ShapeDtypeStruct((G, NCOORD3), jnp.float32),
    )(h, a, w1, b1, w2, b2, g, bt, batch, wc, bc)


def kernel(x, edge_index, batch, emb_table, W1_0, b1_0, W2_0, b2_0, gamma_0,
           beta_0, W1_1, b1_1, W2_1, b2_1, gamma_1, beta_1, W1_2, b1_2, W2_2,
           b2_2, gamma_2, beta_2, Wc, bc):
    zeros = jnp.zeros((ROWS_PER_TILE, EMB), jnp.float32)
    batch2 = batch.reshape(N, 1)

    h = _emb_gather(x.reshape(N), emb_table)

    params = [
        (W1_0, b1_0.reshape(1, EMB), W2_0, b2_0.reshape(1, EMB),
         gamma_0.reshape(1, EMB), beta_0.reshape(1, EMB)),
        (W1_1, b1_1.reshape(1, EMB), W2_1, b2_1.reshape(1, EMB),
         gamma_1.reshape(1, EMB), beta_1.reshape(1, EMB)),
        (W1_2, b1_2.reshape(1, EMB), W2_2, b2_2.reshape(1, EMB),
         gamma_2.reshape(1, EMB), beta_2.reshape(1, EMB)),
    ]

    for l, (w1, b1, w2, b2, g, bt) in enumerate(params):
        agg = _edge_agg(edge_index, h, zeros)
        if l < 2:
            h = _mlp_call(h, agg, w1, b1, w2, b2, g, bt, leaky=True)
        else:
            coords = _final_call(h, agg, w1, b1, w2, b2, g, bt,
                                 batch2, Wc, bc.reshape(1, NCOORD3))
    return coords.reshape(-1, 3)
